# Initial kernel scaffold; baseline (speedup 1.0000x reference)
import jax
import jax.numpy as jnp
from jax.experimental import pallas as pl


def kernel(feat, edge_index, node_type, W_self, b_self, W_neigh, b_neigh):
    raise NotImplementedError("write your pallas kernel here")


# speedup vs baseline: 4.2777x; 4.2777x over previous
import jax
import jax.numpy as jnp
from jax import lax
from jax.experimental import pallas as pl
from jax.experimental.pallas import tpu as pltpu
from jax.experimental.pallas import tpu_sc as plsc

N = 10000
D = 128
N_PAD = 10112
NC = 2
NS = 16
NW = NC * NS
C = 64
CH = 160
H = 2
CHH = CH // H
EPT = C * CH
E_PAD = NW * EPT
ROWS_PER_TILE = N_PAD // NS
DR = 80
DEG_OFF = 10016
DST_PAD = 10008


def _sc_scatter_body(feat_hbm, src_hbm, dst_hbm, out_hbm,
                     acc, srcv, dstv, deg_pr, buf, sem0, sem1):
    cid = lax.axis_index("c")
    sid = lax.axis_index("s")
    wid = cid * NS + sid

    zv = jnp.zeros((16,), jnp.float32)

    def zero_buf0(r, _):
        for k in range(D // 16):
            buf[0, r, pl.ds(k * 16, 16)] = zv
        return 0

    lax.fori_loop(0, C, zero_buf0, 0)

    def zero_deg(r, _):
        for k in range(D // 16):
            deg_pr[r, pl.ds(k * 16, 16)] = zv
        return 0

    lax.fori_loop(0, DR, zero_deg, 0)

    base = sid * ROWS_PER_TILE
    for k in range(ROWS_PER_TILE // C):
        pltpu.sync_copy(buf.at[0], acc.at[pl.ds(base + k * C, C)])
    rem = ROWS_PER_TILE % C
    if rem:
        pltpu.sync_copy(buf.at[0, pl.ds(0, rem)],
                        acc.at[pl.ds(base + ROWS_PER_TILE - rem, rem)])

    plsc.subcore_barrier()

    ones16 = jnp.ones((16,), jnp.float32)

    def count_deg(c):
        for k in range(C // 16):
            d16 = dstv[c, pl.ds(k * 16, 16)]
            plsc.addupdate_scatter(
                deg_pr, [lax.shift_right_logical(d16, 7),
                         lax.bitwise_and(d16, 127)],
                ones16)

    for h in range(H):
        pltpu.sync_copy(src_hbm.at[wid, h], srcv)
        pltpu.sync_copy(dst_hbm.at[wid, h], dstv)
        pltpu.async_copy(feat_hbm.at[srcv.at[0]], buf.at[0], sem0)

        def step(g, _):
            c0 = 2 * g
            c1 = c0 + 1
            pltpu.async_copy(feat_hbm.at[srcv.at[c1]], buf.at[1], sem1)
            count_deg(c0)
            pltpu.make_async_copy(feat_hbm.at[srcv.at[c0]], buf.at[0], sem0).wait()
            pltpu.sync_copy(buf.at[0], acc.at[dstv.at[c0]], add=True)

            @pl.when(g < CHH // 2 - 1)
            def _():
                pltpu.async_copy(feat_hbm.at[srcv.at[c0 + 2]], buf.at[0], sem0)

            count_deg(c1)
            pltpu.make_async_copy(feat_hbm.at[srcv.at[c1]], buf.at[1], sem1).wait()
            pltpu.sync_copy(buf.at[1], acc.at[dstv.at[c1]], add=True)
            return 0

        lax.fori_loop(0, CHH // 2, step, 0)

    iota16 = lax.iota(jnp.int32, 16)
    for k in range(DR // 16):
        pltpu.sync_copy(deg_pr.at[pl.ds(k * 16, 16)],
                        acc.at[DEG_OFF + k * 16 + iota16], add=True)

    plsc.subcore_barrier()
    pltpu.sync_copy(acc.at[pl.ds(base, ROWS_PER_TILE)],
                    out_hbm.at[cid, pl.ds(base, ROWS_PER_TILE)])


_sc_scatter = pl.kernel(
    _sc_scatter_body,
    out_type=jax.ShapeDtypeStruct((NC, N_PAD, D), jnp.float32),
    mesh=plsc.VectorSubcoreMesh(core_axis_name="c", subcore_axis_name="s"),
    compiler_params=pltpu.CompilerParams(needs_layout_passes=False),
    scratch_types=[
        pltpu.VMEM_SHARED((N_PAD, D), jnp.float32),
        pltpu.VMEM((CHH, C), jnp.int32),
        pltpu.VMEM((CHH, C), jnp.int32),
        pltpu.VMEM((DR, D), jnp.float32),
        pltpu.VMEM((2, C, D), jnp.float32),
        pltpu.SemaphoreType.DMA,
        pltpu.SemaphoreType.DMA,
    ],
)


BN = 1024
N_OUT = 10240


def _tc_epilogue_body(parts_ref, deg_ref, feat_ref, wsT_ref, wnT_ref,
                      bs_ref, bn_ref, out_ref):
    p = parts_ref[...]
    summed = p[0] + p[1]
    dp = deg_ref[...]
    deg = dp[0] + dp[1]
    scale = 1.0 / jnp.maximum(deg, 1.0)
    x = feat_ref[...]
    out_ref[...] = (
        jnp.dot(x, wsT_ref[...], preferred_element_type=jnp.float32)
        + bs_ref[...]
        + scale * jnp.dot(summed, wnT_ref[...], preferred_element_type=jnp.float32)
        + bn_ref[...]
    )


def _tc_epilogue(parts, deg, feat_pad, wsT, wnT, bs, bn):
    return pl.pallas_call(
        _tc_epilogue_body,
        grid=(N_OUT // BN,),
        in_specs=[
            pl.BlockSpec((NC, BN, D), lambda i: (0, i, 0)),
            pl.BlockSpec((NC, BN, 1), lambda i: (0, i, 0)),
            pl.BlockSpec((BN, D), lambda i: (i, 0)),
            pl.BlockSpec((D, D), lambda i: (0, 0)),
            pl.BlockSpec((D, D), lambda i: (0, 0)),
            pl.BlockSpec((1, D), lambda i: (0, 0)),
            pl.BlockSpec((1, D), lambda i: (0, 0)),
        ],
        out_specs=pl.BlockSpec((BN, D), lambda i: (i, 0)),
        out_shape=jax.ShapeDtypeStruct((N_OUT, D), jnp.float32),
    )(parts, deg, feat_pad, wsT, wnT, bs, bn)


def kernel(feat, edge_index, node_type, W_self, b_self, W_neigh, b_neigh):
    del node_type
    E = edge_index.shape[1]
    src = edge_index[0].astype(jnp.int32)
    dst = edge_index[1].astype(jnp.int32)
    pad = E_PAD - E
    src_p = jnp.concatenate([src, jnp.zeros((pad,), jnp.int32)])
    dst_p = jnp.concatenate([dst, jnp.full((pad,), DST_PAD, jnp.int32)])
    src3 = src_p.reshape(NW, H, CHH, C)
    dst3 = dst_p.reshape(NW, H, CHH, C)

    parts = _sc_scatter(feat, src3, dst3)

    deg = parts[:, DEG_OFF:DEG_OFF + DR, :].reshape(NC, DR * D, 1)
    feat_pad = jnp.pad(feat, ((0, N_OUT - N), (0, 0)))
    out = _tc_epilogue(parts, deg, feat_pad, W_self.T, W_neigh.T,
                       b_self[None, :], b_neigh[None, :])
    return out[:N]
